# baseline (speedup 1.0000x reference)
import functools

import jax
import jax.numpy as jnp
from jax import lax
from jax.experimental import pallas as pl
from jax.experimental.pallas import tpu as pltpu
from jax.experimental.pallas import tpu_sc as plsc

B, L, D = 16384, 200, 32
DP = 40
V = 1000000
NC, NS = 2, 16
NW = NC * NS
B_PER_W = B // NW
HALF = 256
CT, CS = D // 8, 8
BT_L = HALF // 128
NBT = B // 128
KB = 1600
NBLK = V // KB
SLOTS = -(-NBLK // NW)


@functools.partial(
    pl.kernel,
    mesh=plsc.VectorSubcoreMesh(core_axis_name="c", subcore_axis_name="s"),
    out_type=jax.ShapeDtypeStruct((V, DP), jnp.float32),
    compiler_params=pltpu.CompilerParams(use_tc_tiling_on_sc=False,
                                         needs_layout_passes=False),
    scratch_types=[
        pltpu.VMEM((D, KB + 1), jnp.float32),
        pltpu.VMEM((KB, DP), jnp.float32),
        pltpu.SemaphoreType.DMA,
    ],
)
def _prep(tt_hbm, out_hbm, tin, tout, lsem):
    wid = lax.axis_index("s") * NC + lax.axis_index("c")
    iota = lax.iota(jnp.int32, 16)

    for i in range(SLOTS):
        blk = jnp.minimum(wid + NW * i, NBLK - 1)
        r0 = pl.multiple_of(blk * KB, 8)
        for c in range(D):
            pltpu.async_copy(tt_hbm.at[c, pl.ds(r0, KB)],
                             tin.at[c, pl.ds(0, KB)], lsem)
        for c in range(D):
            pltpu.make_async_copy(tt_hbm.at[c, pl.ds(0, KB)],
                                  tin.at[c, pl.ds(0, KB)], lsem).wait()

        @plsc.parallel_loop(0, KB, unroll=4)
        def row_body(r):
            rcol = jnp.full((16,), 0, jnp.int32) + r
            tout[r, pl.ds(0, 16)] = plsc.load_gather(tin, [iota, rcol])
            tout[r, pl.ds(16, 16)] = plsc.load_gather(tin, [iota + 16, rcol])

        pltpu.sync_copy(tout, out_hbm.at[pl.ds(r0, KB)])


@functools.partial(
    pl.kernel,
    mesh=plsc.VectorSubcoreMesh(core_axis_name="c", subcore_axis_name="s"),
    out_type=jax.ShapeDtypeStruct((L, CT, NBT, CS, 128), jnp.float32),
    compiler_params=pltpu.CompilerParams(use_tc_tiling_on_sc=False,
                                         needs_layout_passes=False),
    scratch_types=(
        [pltpu.VMEM((HALF, L + 1), jnp.int32)]
        + [pltpu.VMEM((HALF,), jnp.int32) for _ in range(2)]
        + [pltpu.VMEM((HALF, DP), jnp.float32) for _ in range(2)]
        + [pltpu.VMEM((CT, BT_L, CS, 128), jnp.float32) for _ in range(2)]
        + [pltpu.VMEM((L, D), jnp.float32)]
        + [pltpu.SemaphoreType.DMA for _ in range(4)]
    ),
)
def _embed(x_hbm, tab_hbm, pos_hbm, out_hbm, xblk, idx0, idx1, rows0, rows1,
           slab0, slab1, pos_v, gsem0, gsem1, ssem0, ssem1):
    idx_b = (idx0, idx1)
    rows_b = (rows0, rows1)
    slab_b = (slab0, slab1)
    gsem_b = (gsem0, gsem1)
    ssem_b = (ssem0, ssem1)
    wid = lax.axis_index("s") * NC + lax.axis_index("c")
    iota = lax.iota(jnp.int32, 16)
    zeros16 = jnp.full((16,), 0, jnp.int32)

    pltpu.sync_copy(pos_hbm, pos_v)

    def build_idx(l, lb):
        col = zeros16 + l
        for j in range(HALF // 16):
            idx_b[lb][pl.ds(16 * j, 16)] = plsc.load_gather(
                xblk, [iota + (16 * j), col])

    def fire_gather(lb):
        for off in range(0, HALF, 128):
            pltpu.async_copy(tab_hbm.at[idx_b[lb].at[pl.ds(off, 128)]],
                             rows_b[lb].at[pl.ds(off, 128)], gsem_b[lb])

    def wait_gather(lb):
        for off in range(0, HALF, 128):
            pltpu.make_async_copy(
                tab_hbm.at[idx_b[lb].at[pl.ds(off, 128)]],
                rows_b[lb].at[pl.ds(off, 128)], gsem_b[lb]).wait()

    def fire_store(l, h, lb):
        wbt0 = wid * (B_PER_W // 128) + h * BT_L
        pltpu.async_copy(slab_b[lb], out_hbm.at[l, :, pl.ds(wbt0, BT_L)],
                         ssem_b[lb])

    def wait_store(lb):
        pltpu.make_async_copy(slab_b[lb], out_hbm.at[0, :, pl.ds(0, BT_L)],
                              ssem_b[lb]).wait()

    def transpose_add(l, lb):
        rows = rows_b[lb]
        slab = slab_b[lb]
        lvec = zeros16 + l

        @plsc.parallel_loop(0, D, unroll=4)
        def col_body(c):
            ct = c // CS
            cs = c % CS
            ccol = zeros16 + c
            pvec = plsc.load_gather(pos_v, [lvec, ccol])
            for btl in range(BT_L):
                for k in range(8):
                    ridx = iota + (btl * 128 + k * 16)
                    v = plsc.load_gather(rows, [ridx, ccol]) + pvec
                    slab[ct, btl, cs, pl.ds(k * 16, 16)] = v

    for h in range(2):
        b0 = wid * B_PER_W + h * HALF
        pltpu.sync_copy(x_hbm.at[pl.ds(b0, HALF), :], xblk.at[:, pl.ds(0, L)])
        build_idx(0, 0)
        fire_gather(0)
        build_idx(1, 1)
        fire_gather(1)
        for l0 in range(2):
            wait_gather(l0)
            transpose_add(l0, l0)
            fire_store(l0, h, l0)
            build_idx(l0 + 2, l0)
            fire_gather(l0)

        def steady(l2, carry):
            for lb in range(2):
                l = 2 * l2 + lb
                wait_gather(lb)
                wait_store(lb)
                transpose_add(l, lb)
                fire_store(l, h, lb)
                build_idx(l + 2, lb)
                fire_gather(lb)
            return carry

        lax.fori_loop(1, (L - 4) // 2 + 1, steady, 0)

        for l0, lb in ((L - 2, 0), (L - 1, 1)):
            wait_gather(lb)
            wait_store(lb)
            transpose_add(l0, lb)
            fire_store(l0, h, lb)
        wait_store(0)
        wait_store(1)


def kernel(x, window_table, pos_table):
    table_p = _prep(window_table.T)
    g5 = _embed(x.astype(jnp.int32), table_p, pos_table)
    return g5.transpose(2, 4, 0, 1, 3).reshape(B, L, D)


# speedup vs baseline: 2.9341x; 2.9341x over previous
import functools

import jax
import jax.numpy as jnp
from jax import lax
from jax.experimental import pallas as pl
from jax.experimental.pallas import tpu as pltpu
from jax.experimental.pallas import tpu_sc as plsc

B, L, D = 16384, 200, 32
DP = 40
V = 1000000
NC, NS = 2, 16
NW = NC * NS
B_PER_W = B // NW
HALF = 256
CT, CS = D // 8, 8
BT_L = HALF // 128
NBT = B // 128
KB = 1536
NBLK = V // KB
SLOTS = -(-NBLK // NW)


@functools.partial(
    pl.kernel,
    mesh=plsc.VectorSubcoreMesh(core_axis_name="c", subcore_axis_name="s"),
    out_type=jax.ShapeDtypeStruct((V, DP), jnp.float32),
    compiler_params=pltpu.CompilerParams(use_tc_tiling_on_sc=False,
                                         needs_layout_passes=False),
    scratch_types=[
        pltpu.VMEM((KB, D), jnp.float32),
        pltpu.VMEM((KB, D), jnp.float32),
        pltpu.SemaphoreType.DMA,
        pltpu.SemaphoreType.DMA,
        pltpu.SemaphoreType.DMA,
        pltpu.SemaphoreType.DMA,
    ],
)
def _prep(tab_hbm, out_hbm, tin0, tin1, isem0, isem1, osem0, osem1):
    tin_b = (tin0, tin1)
    isem_b = (isem0, isem1)
    osem_b = (osem0, osem1)
    wid = lax.axis_index("s") * NC + lax.axis_index("c")

    def blk_start(i):
        blk = jnp.minimum(wid + NW * i, NBLK - 1)
        return pl.multiple_of(blk * KB, 8)

    def fire_in(i, b):
        pltpu.async_copy(tab_hbm.at[pl.ds(blk_start(i), KB)], tin_b[b],
                         isem_b[b])

    def wait_in(b):
        pltpu.make_async_copy(tab_hbm.at[pl.ds(0, KB)], tin_b[b],
                              isem_b[b]).wait()

    def fire_out(i, b):
        pltpu.async_copy(tin_b[b],
                         out_hbm.at[pl.ds(blk_start(i), KB), pl.ds(0, D)],
                         osem_b[b])

    def wait_out(b):
        pltpu.make_async_copy(tin_b[b],
                              out_hbm.at[pl.ds(0, KB), pl.ds(0, D)],
                              osem_b[b]).wait()

    fire_in(0, 0)
    for i in range(SLOTS):
        b = i % 2
        if i + 1 < SLOTS:
            fire_in(i + 1, 1 - b)
        wait_in(b)
        if i >= 2:
            wait_out(b)
        fire_out(i, b)
    wait_out(0)
    wait_out(1)
    TKB = V - NBLK * KB
    if TKB:
        pltpu.sync_copy(tab_hbm.at[pl.ds(NBLK * KB, TKB)],
                        tin0.at[pl.ds(0, TKB)])
        pltpu.sync_copy(tin0.at[pl.ds(0, TKB)],
                        out_hbm.at[pl.ds(NBLK * KB, TKB), pl.ds(0, D)])


@functools.partial(
    pl.kernel,
    mesh=plsc.VectorSubcoreMesh(core_axis_name="c", subcore_axis_name="s"),
    out_type=jax.ShapeDtypeStruct((L, CT, NBT, CS, 128), jnp.float32),
    compiler_params=pltpu.CompilerParams(use_tc_tiling_on_sc=False,
                                         needs_layout_passes=False),
    scratch_types=(
        [pltpu.VMEM((HALF, L + 1), jnp.int32)]
        + [pltpu.VMEM((HALF,), jnp.int32) for _ in range(2)]
        + [pltpu.VMEM((HALF, DP), jnp.float32) for _ in range(2)]
        + [pltpu.VMEM((CT, BT_L, CS, 128), jnp.float32) for _ in range(2)]
        + [pltpu.VMEM((L, D), jnp.float32)]
        + [pltpu.SemaphoreType.DMA for _ in range(4)]
    ),
)
def _embed(x_hbm, tab_hbm, pos_hbm, out_hbm, xblk, idx0, idx1, rows0, rows1,
           slab0, slab1, pos_v, gsem0, gsem1, ssem0, ssem1):
    idx_b = (idx0, idx1)
    rows_b = (rows0, rows1)
    slab_b = (slab0, slab1)
    gsem_b = (gsem0, gsem1)
    ssem_b = (ssem0, ssem1)
    wid = lax.axis_index("s") * NC + lax.axis_index("c")
    iota = lax.iota(jnp.int32, 16)
    zeros16 = jnp.full((16,), 0, jnp.int32)

    pltpu.sync_copy(pos_hbm, pos_v)

    def build_idx(l, lb):
        col = zeros16 + l
        for j in range(HALF // 16):
            idx_b[lb][pl.ds(16 * j, 16)] = plsc.load_gather(
                xblk, [iota + (16 * j), col])

    def fire_gather(lb):
        for off in range(0, HALF, 128):
            pltpu.async_copy(tab_hbm.at[idx_b[lb].at[pl.ds(off, 128)]],
                             rows_b[lb].at[pl.ds(off, 128)], gsem_b[lb])

    def wait_gather(lb):
        for off in range(0, HALF, 128):
            pltpu.make_async_copy(
                tab_hbm.at[idx_b[lb].at[pl.ds(off, 128)]],
                rows_b[lb].at[pl.ds(off, 128)], gsem_b[lb]).wait()

    def fire_store(l, h, lb):
        wbt0 = wid * (B_PER_W // 128) + h * BT_L
        pltpu.async_copy(slab_b[lb], out_hbm.at[l, :, pl.ds(wbt0, BT_L)],
                         ssem_b[lb])

    def wait_store(lb):
        pltpu.make_async_copy(slab_b[lb], out_hbm.at[0, :, pl.ds(0, BT_L)],
                              ssem_b[lb]).wait()

    def transpose_add(l, lb):
        rows = rows_b[lb]
        slab = slab_b[lb]
        lvec = zeros16 + l

        @plsc.parallel_loop(0, D, unroll=4)
        def col_body(c):
            ct = c // CS
            cs = c % CS
            ccol = zeros16 + c
            pvec = plsc.load_gather(pos_v, [lvec, ccol])
            for btl in range(BT_L):
                for k in range(8):
                    ridx = iota + (btl * 128 + k * 16)
                    v = plsc.load_gather(rows, [ridx, ccol]) + pvec
                    slab[ct, btl, cs, pl.ds(k * 16, 16)] = v

    for h in range(2):
        b0 = wid * B_PER_W + h * HALF
        pltpu.sync_copy(x_hbm.at[pl.ds(b0, HALF), :], xblk.at[:, pl.ds(0, L)])
        build_idx(0, 0)
        fire_gather(0)
        build_idx(1, 1)
        fire_gather(1)
        for l0 in range(2):
            wait_gather(l0)
            transpose_add(l0, l0)
            fire_store(l0, h, l0)
            build_idx(l0 + 2, l0)
            fire_gather(l0)

        def steady(l2, carry):
            for lb in range(2):
                l = 2 * l2 + lb
                wait_gather(lb)
                wait_store(lb)
                transpose_add(l, lb)
                fire_store(l, h, lb)
                build_idx(l + 2, lb)
                fire_gather(lb)
            return carry

        lax.fori_loop(1, (L - 4) // 2 + 1, steady, 0)

        for l0, lb in ((L - 2, 0), (L - 1, 1)):
            wait_gather(lb)
            wait_store(lb)
            transpose_add(l0, lb)
            fire_store(l0, h, lb)
        wait_store(0)
        wait_store(1)


def kernel(x, window_table, pos_table):
    table_p = _prep(window_table)
    g5 = _embed(x.astype(jnp.int32), table_p, pos_table)
    return g5.transpose(2, 4, 0, 1, 3).reshape(B, L, D)
